# Initial kernel scaffold; baseline (speedup 1.0000x reference)
import jax
import jax.numpy as jnp
from jax.experimental import pallas as pl


def kernel(x, params, edge_index):
    raise NotImplementedError("write your pallas kernel here")


# speedup vs baseline: 1.7073x; 1.7073x over previous
import functools

import jax
import jax.numpy as jnp
from jax import lax
from jax.experimental import pallas as pl
from jax.experimental.pallas import tpu as pltpu
from jax.experimental.pallas import tpu_sc as plsc

N_NODES = 10000
LAT = 64
N_UNSHARED = 4


_NW = 32


def _make_sc_gather(E, D, chunk):
    n_iters = (E // _NW) // chunk
    mesh = plsc.VectorSubcoreMesh(core_axis_name="c", subcore_axis_name="s")

    @functools.partial(
        pl.kernel,
        mesh=mesh,
        out_type=jax.ShapeDtypeStruct((E, D), jnp.float32),
        scratch_types=[
            pltpu.VMEM((chunk,), jnp.int32),
            pltpu.VMEM((chunk, D), jnp.float32),
            pltpu.SemaphoreType.DMA,
        ],
    )
    def gather_kernel(table_hbm, idx_hbm, out_hbm, idx_v, rows_v, sem):
        wid = lax.axis_index("s") * 2 + lax.axis_index("c")
        base = wid * (E // _NW)

        def body(i, carry):
            off = base + i * chunk
            pltpu.sync_copy(idx_hbm.at[pl.ds(off, chunk)], idx_v)
            pltpu.async_copy(table_hbm.at[idx_v], rows_v, sem).wait()
            pltpu.sync_copy(rows_v, out_hbm.at[pl.ds(off, chunk)])
            return carry

        lax.fori_loop(0, n_iters, body, 0)

    return gather_kernel


def _sc_gather(table, idx, chunk):
    return _make_sc_gather(idx.shape[0], table.shape[1], chunk)(table, idx)


def _ln(h, g, b):
    m = jnp.mean(h, axis=-1, keepdims=True)
    v = jnp.mean((h - m) ** 2, axis=-1, keepdims=True)
    return (h - m) / jnp.sqrt(v + 1e-5) * g + b


def _edge_mlp_kernel(xi_ref, xj_ref, w1_ref, b1_ref, g1_ref, be1_ref,
                     w2_ref, b2_ref, g2_ref, be2_ref, w3_ref, b3_ref,
                     out_ref):
    xi = xi_ref[...]
    xj = xj_ref[...]
    d = xj[:, :3] - xi[:, :3]
    dist2 = jnp.sum(d * d, axis=-1, keepdims=True)
    pad = jnp.zeros((xi.shape[0], 2), jnp.float32)
    inp = jnp.concatenate([xi, xj[:, 3:], dist2, pad], axis=-1)
    for u in range(N_UNSHARED):
        h = jnp.dot(inp, w1_ref[u], preferred_element_type=jnp.float32)
        h = jax.nn.relu(_ln(h + b1_ref[u], g1_ref[u], be1_ref[u]))
        h = jnp.dot(h, w2_ref[u], preferred_element_type=jnp.float32)
        h = jax.nn.relu(_ln(h + b2_ref[u], g2_ref[u], be2_ref[u]))
        h = jnp.dot(h, w3_ref[u], preferred_element_type=jnp.float32)
        out_ref[:, u * LAT:(u + 1) * LAT] = h + b3_ref[u]


def _edge_mlp(xi, xj, plist, eb):
    E, C = xi.shape
    dp = 2 * C
    w1 = jnp.stack([jnp.pad(p['W1'], ((0, 2), (0, 0))) for p in plist])
    b1 = jnp.stack([p['b1'] for p in plist])
    g1 = jnp.stack([p['g1'] for p in plist])
    be1 = jnp.stack([p['be1'] for p in plist])
    w2 = jnp.stack([p['W2'] for p in plist])
    b2 = jnp.stack([p['b2'] for p in plist])
    g2 = jnp.stack([p['g2'] for p in plist])
    be2 = jnp.stack([p['be2'] for p in plist])
    w3 = jnp.stack([p['W3'] for p in plist])
    b3 = jnp.stack([p['b3'] for p in plist])

    full = lambda s: pl.BlockSpec(s, lambda i: tuple(0 for _ in s))
    grid = (E // eb,)
    return pl.pallas_call(
        _edge_mlp_kernel,
        grid=grid,
        in_specs=[
            pl.BlockSpec((eb, C), lambda i: (i, 0)),
            pl.BlockSpec((eb, C), lambda i: (i, 0)),
            full((N_UNSHARED, dp, LAT)),
            full((N_UNSHARED, LAT)),
            full((N_UNSHARED, LAT)),
            full((N_UNSHARED, LAT)),
            full((N_UNSHARED, LAT, LAT)),
            full((N_UNSHARED, LAT)),
            full((N_UNSHARED, LAT)),
            full((N_UNSHARED, LAT)),
            full((N_UNSHARED, LAT, LAT)),
            full((N_UNSHARED, LAT)),
        ],
        out_specs=pl.BlockSpec((eb, N_UNSHARED * LAT), lambda i: (i, 0)),
        out_shape=jax.ShapeDtypeStruct((E, N_UNSHARED * LAT), jnp.float32),
    )(xi, xj, w1, b1, g1, be1, w2, b2, g2, be2, w3, b3)


def _final_kernel(xr_ref, x_ref,
                  fw1, fb1, fg1, fbe1, fw2, fb2, fg2, fbe2, fw3, fb3,
                  gw1, gb1, gg1, gbe1, gw2, gb2, gg2, gbe2, gw3, gb3,
                  out_ref):
    def mlp(inp, w1, b1, g1, be1, w2, b2, g2, be2, w3, b3):
        h = jnp.dot(inp, w1[...], preferred_element_type=jnp.float32)
        h = jax.nn.relu(_ln(h + b1[...], g1[...], be1[...]))
        h = jnp.dot(h, w2[...], preferred_element_type=jnp.float32)
        h = jax.nn.relu(_ln(h + b2[...], g2[...], be2[...]))
        return jnp.dot(h, w3[...], preferred_element_type=jnp.float32) + b3[...]

    a = mlp(xr_ref[...], fw1, fb1, fg1, fbe1, fw2, fb2, fg2, fbe2, fw3, fb3)
    b = mlp(x_ref[...], gw1, gb1, gg1, gbe1, gw2, gb2, gg2, gbe2, gw3, gb3)
    out_ref[...] = a + b


def _final_mlp(xr, x, fc, ghm, nb):
    N, D1 = xr.shape
    C = x.shape[1]
    dout = 128

    def padded(p):
        w3 = jnp.pad(p['W3'], ((0, 0), (0, dout - p['W3'].shape[1])))
        b3 = jnp.pad(p['b3'], (0, dout - p['b3'].shape[0]))
        return w3, b3

    fw3, fb3 = padded(fc)
    gw3, gb3 = padded(ghm)
    full = lambda s: pl.BlockSpec(s, lambda i: tuple(0 for _ in s))
    args = [
        fc['W1'], fc['b1'], fc['g1'], fc['be1'], fc['W2'], fc['b2'],
        fc['g2'], fc['be2'], fw3, fb3,
        ghm['W1'], ghm['b1'], ghm['g1'], ghm['be1'], ghm['W2'], ghm['b2'],
        ghm['g2'], ghm['be2'], gw3, gb3,
    ]
    out = pl.pallas_call(
        _final_kernel,
        grid=(N // nb,),
        in_specs=[pl.BlockSpec((nb, D1), lambda i: (i, 0)),
                  pl.BlockSpec((nb, C), lambda i: (i, 0))]
        + [full(a.shape) for a in args],
        out_specs=pl.BlockSpec((nb, dout), lambda i: (i, 0)),
        out_shape=jax.ShapeDtypeStruct((N, dout), jnp.float32),
    )(xr, x, *args)
    return out[:, :4]


def _aggregate(m, dst, cnt, pos_mask):
    s = jax.ops.segment_sum(m, dst, num_segments=N_NODES)
    mx = jax.ops.segment_max(m, dst, num_segments=N_NODES)
    mx = jnp.where(pos_mask, mx, 0.0)
    mean = s / cnt
    parts = []
    for u in range(N_UNSHARED):
        sl = slice(u * LAT, (u + 1) * LAT)
        parts.extend([s[:, sl], mx[:, sl], mean[:, sl]])
    return jnp.concatenate(parts, axis=-1)


def kernel(x, params, edge_index):
    src = edge_index[0]
    dst = edge_index[1]

    cnt = jax.ops.segment_sum(jnp.ones((src.shape[0], 1), jnp.float32), dst,
                              num_segments=N_NODES)
    pos_mask = cnt > 0
    cnt = jnp.maximum(cnt, 1.0)

    xi = _sc_gather(x, dst, 200)
    xj = _sc_gather(x, src, 200)
    m0 = _edge_mlp(xi, xj, params['g0'], 2000)
    h = _aggregate(m0, dst, cnt, pos_mask)

    xr = jax.nn.relu(h)
    xi2 = _sc_gather(xr, dst, 40)
    xj2 = _sc_gather(xr, src, 40)
    m1 = _edge_mlp(xi2, xj2, params['g1'], 2000)
    h = h + _aggregate(m1, dst, cnt, pos_mask)

    xr2 = jax.nn.relu(h)
    return _final_mlp(xr2, x, params['fc'], params['ghm'], 2000)
